# initial kernel scaffold (unmeasured)
import jax
import jax.numpy as jnp
from jax import lax
from jax.experimental import pallas as pl
from jax.experimental.pallas import tpu as pltpu

N_DEV = 4
M_PER = 1024
K = 4096
N_PER = 2048
HALF = M_PER // 2


def _body(x_ref, w_ref, s_ref, out_ref, xg, send_sems, recv_sems):
    my = lax.axis_index("i")
    left = (my + N_DEV - 1) % N_DEV
    right = (my + 1) % N_DEV
    opp = (my + 2) % N_DEV

    barrier = pltpu.get_barrier_semaphore()
    for nbr in (left, right):
        pl.semaphore_signal(
            barrier, inc=1,
            device_id=(nbr,), device_id_type=pl.DeviceIdType.MESH,
        )
    pl.semaphore_wait(barrier, 2)

    xg[my] = x_ref[...]

    full_r = pltpu.make_async_remote_copy(
        src_ref=x_ref, dst_ref=xg.at[my],
        send_sem=send_sems.at[0], recv_sem=recv_sems.at[0],
        device_id=(right,), device_id_type=pl.DeviceIdType.MESH,
    )
    full_l = pltpu.make_async_remote_copy(
        src_ref=x_ref, dst_ref=xg.at[my],
        send_sem=send_sems.at[1], recv_sem=recv_sems.at[1],
        device_id=(left,), device_id_type=pl.DeviceIdType.MESH,
    )
    full_r.start()
    full_l.start()

    recv_from_left = pltpu.make_async_remote_copy(
        src_ref=x_ref, dst_ref=xg.at[left],
        send_sem=send_sems.at[0], recv_sem=recv_sems.at[0],
        device_id=(left,), device_id_type=pl.DeviceIdType.MESH,
    )
    recv_from_right = pltpu.make_async_remote_copy(
        src_ref=x_ref, dst_ref=xg.at[right],
        send_sem=send_sems.at[1], recv_sem=recv_sems.at[1],
        device_id=(right,), device_id_type=pl.DeviceIdType.MESH,
    )

    recv_from_left.wait_recv()
    fwd_r = pltpu.make_async_remote_copy(
        src_ref=xg.at[left, pl.ds(0, HALF)],
        dst_ref=xg.at[left, pl.ds(0, HALF)],
        send_sem=send_sems.at[2], recv_sem=recv_sems.at[2],
        device_id=(right,), device_id_type=pl.DeviceIdType.MESH,
    )
    fwd_r.start()

    recv_from_right.wait_recv()
    fwd_l = pltpu.make_async_remote_copy(
        src_ref=xg.at[right, pl.ds(HALF, HALF)],
        dst_ref=xg.at[right, pl.ds(HALF, HALF)],
        send_sem=send_sems.at[3], recv_sem=recv_sems.at[3],
        device_id=(left,), device_id_type=pl.DeviceIdType.MESH,
    )
    fwd_l.start()

    recv_half_a = pltpu.make_async_remote_copy(
        src_ref=xg.at[opp, pl.ds(0, HALF)],
        dst_ref=xg.at[opp, pl.ds(0, HALF)],
        send_sem=send_sems.at[2], recv_sem=recv_sems.at[2],
        device_id=(left,), device_id_type=pl.DeviceIdType.MESH,
    )
    recv_half_b = pltpu.make_async_remote_copy(
        src_ref=xg.at[opp, pl.ds(HALF, HALF)],
        dst_ref=xg.at[opp, pl.ds(HALF, HALF)],
        send_sem=send_sems.at[3], recv_sem=recv_sems.at[3],
        device_id=(right,), device_id_type=pl.DeviceIdType.MESH,
    )
    recv_half_a.wait_recv()
    recv_half_b.wait_recv()

    s = s_ref[0]
    for off in range(N_DEV):
        j = (my + off) % N_DEV
        acc = jnp.dot(xg[j], w_ref[...], preferred_element_type=jnp.float32)
        y = acc * s
        z = y * (1.0 / (1.0 + jnp.exp(-jnp.clip(y, -60.0, 60.0))))
        out_ref[pl.ds(j * M_PER, M_PER), :] = z

    full_r.wait_send()
    full_l.wait_send()
    fwd_r.wait_send()
    fwd_l.wait_send()


def kernel(x, w_mat, scale_x, scale_w):
    my = lax.axis_index("i")

    xq = x if x.dtype == jnp.float8_e4m3fn else x.astype(jnp.float8_e4m3fn)
    w_slice = lax.dynamic_slice(w_mat, (0, my * N_PER), (K, N_PER))
    wq = (
        w_slice
        if w_slice.dtype == jnp.float8_e5m2
        else w_slice.astype(jnp.float8_e5m2)
    )
    s = (scale_x * scale_w).astype(jnp.float32)

    return pl.pallas_call(
        _body,
        out_shape=jax.ShapeDtypeStruct((N_DEV * M_PER, N_PER), jnp.float32),
        in_specs=[
            pl.BlockSpec(memory_space=pltpu.VMEM),
            pl.BlockSpec(memory_space=pltpu.VMEM),
            pl.BlockSpec(memory_space=pltpu.SMEM),
        ],
        out_specs=pl.BlockSpec(memory_space=pltpu.VMEM),
        scratch_shapes=[
            pltpu.VMEM((N_DEV, M_PER, K), jnp.float8_e4m3fn),
            pltpu.SemaphoreType.DMA((4,)),
            pltpu.SemaphoreType.DMA((4,)),
        ],
        compiler_params=pltpu.CompilerParams(collective_id=0),
    )(xq, wq, s)


# baseline (device time: 112767 ns/iter reference)
import jax
import jax.numpy as jnp
from jax import lax
from jax.experimental import pallas as pl
from jax.experimental.pallas import tpu as pltpu

N_DEV = 4
M_PER = 1024
K = 4096
N_PER = 2048
HALF = M_PER // 2
QUART = M_PER // 4
WCHUNK = 512

SLOT_L = 0
SLOT_R = 1
SLOT_O = 2


def _body(x_ref, w_hbm, s_ref, out_hbm,
          xg, wq, wstage, ostage, send_sems, recv_sems, wdma_sems, out_sems):
    my = lax.axis_index("i")
    left = (my + N_DEV - 1) % N_DEV
    right = (my + 1) % N_DEV
    opp = (my + 2) % N_DEV

    barrier = pltpu.get_barrier_semaphore()
    for nbr in (left, right):
        pl.semaphore_signal(
            barrier, inc=1,
            device_id=(nbr,), device_id_type=pl.DeviceIdType.MESH,
        )
    pl.semaphore_wait(barrier, 2)

    def rdma(src, dst, si, ri, dev):
        return pltpu.make_async_remote_copy(
            src_ref=src, dst_ref=dst,
            send_sem=send_sems.at[si], recv_sem=recv_sems.at[ri],
            device_id=(dev,), device_id_type=pl.DeviceIdType.MESH,
        )

    a = pl.ds(0, HALF)
    b = pl.ds(HALF, HALF)
    s_myA_r = rdma(x_ref.at[a], xg.at[SLOT_L, a], 0, 0, right)
    s_myB_r = rdma(x_ref.at[b], xg.at[SLOT_L, b], 1, 1, right)
    s_myB_l = rdma(x_ref.at[b], xg.at[SLOT_R, b], 2, 2, left)
    s_myA_l = rdma(x_ref.at[a], xg.at[SLOT_R, a], 3, 3, left)
    s_myA_r.start()
    s_myB_r.start()
    s_myB_l.start()
    s_myA_l.start()

    r_leftA = rdma(x_ref.at[a], xg.at[SLOT_L, a], 0, 0, left)
    r_leftB = rdma(x_ref.at[b], xg.at[SLOT_L, b], 1, 1, left)
    r_rightB = rdma(x_ref.at[b], xg.at[SLOT_R, b], 2, 2, right)
    r_rightA = rdma(x_ref.at[a], xg.at[SLOT_R, a], 3, 3, right)

    q = [pl.ds(i * QUART, QUART) for i in range(4)]
    s_fwd_r = [rdma(xg.at[SLOT_L, q[i]], xg.at[SLOT_O, q[i]], 4 + i, 4 + i, right)
               for i in range(2)]
    s_fwd_l = [rdma(xg.at[SLOT_R, q[i]], xg.at[SLOT_O, q[i]], 6 + (i - 2), 6 + (i - 2), left)
               for i in range(2, 4)]
    r_opp = [rdma(xg.at[SLOT_O, q[i]], xg.at[SLOT_O, q[i]], 4 + i, 4 + i,
                  left if i < 2 else right)
             for i in range(4)]

    n_wchunks = N_PER // WCHUNK
    col0 = my * N_PER
    wdmas = []
    for c in range(min(2, n_wchunks)):
        d = pltpu.make_async_copy(
            w_hbm.at[:, pl.ds(col0 + c * WCHUNK, WCHUNK)],
            wstage.at[c % 2], wdma_sems.at[c % 2],
        )
        d.start()
        wdmas.append(d)
    for c in range(n_wchunks):
        wdmas[c].wait()
        if c + 2 < n_wchunks:
            d = pltpu.make_async_copy(
                w_hbm.at[:, pl.ds(col0 + (c + 2) * WCHUNK, WCHUNK)],
                wstage.at[(c + 2) % 2], wdma_sems.at[(c + 2) % 2],
            )
            d.start()
            wdmas.append(d)
        wq[:, pl.ds(c * WCHUNK, WCHUNK)] = wstage[c % 2].astype(jnp.float8_e5m2)

    s = s_ref[0]
    pending = [None, None]
    state = {"blk": 0}

    def block(x_block, out_row, rows):
        buf = state["blk"] % 2
        state["blk"] += 1
        if pending[buf] is not None:
            pending[buf].wait()
        acc = jnp.dot(x_block, wq[...], preferred_element_type=jnp.float32)
        y = acc * s
        z = y * (1.0 / (1.0 + jnp.exp(-jnp.clip(y, -60.0, 60.0))))
        ostage[buf, pl.ds(0, rows)] = z
        d = pltpu.make_async_copy(
            ostage.at[buf, pl.ds(0, rows)],
            out_hbm.at[pl.ds(out_row, rows)],
            out_sems.at[buf],
        )
        d.start()
        pending[buf] = d

    block(x_ref[0:HALF, :], my * M_PER, HALF)
    block(x_ref[HALF:M_PER, :], my * M_PER + HALF, HALF)

    r_leftA.wait_recv()
    for d in s_fwd_r:
        d.start()
    r_rightB.wait_recv()
    for d in s_fwd_l:
        d.start()
    block(xg[SLOT_L, 0:HALF, :], left * M_PER, HALF)
    block(xg[SLOT_R, HALF:M_PER, :], right * M_PER + HALF, HALF)

    r_leftB.wait_recv()
    block(xg[SLOT_L, HALF:M_PER, :], left * M_PER + HALF, HALF)
    r_rightA.wait_recv()
    block(xg[SLOT_R, 0:HALF, :], right * M_PER, HALF)

    for i in (0, 2, 1, 3):
        r_opp[i].wait_recv()
        block(xg[SLOT_O, i * QUART:(i + 1) * QUART, :],
              opp * M_PER + i * QUART, QUART)

    for d in pending:
        if d is not None:
            d.wait()
    for d in [s_myA_r, s_myB_r, s_myB_l, s_myA_l] + s_fwd_r + s_fwd_l:
        d.wait_send()


def kernel(x, w_mat, scale_x, scale_w):
    xq = x if x.dtype == jnp.float8_e4m3fn else x.astype(jnp.float8_e4m3fn)
    s = (scale_x * scale_w).astype(jnp.float32)

    return pl.pallas_call(
        _body,
        out_shape=jax.ShapeDtypeStruct((N_DEV * M_PER, N_PER), jnp.float32),
        in_specs=[
            pl.BlockSpec(memory_space=pltpu.MemorySpace.VMEM),
            pl.BlockSpec(memory_space=pltpu.MemorySpace.HBM),
            pl.BlockSpec(memory_space=pltpu.MemorySpace.SMEM),
        ],
        out_specs=pl.BlockSpec(memory_space=pltpu.MemorySpace.HBM),
        scratch_shapes=[
            pltpu.VMEM((3, M_PER, K), jnp.float8_e4m3fn),
            pltpu.VMEM((K, N_PER), jnp.float8_e5m2),
            pltpu.VMEM((2, K, WCHUNK), jnp.float32),
            pltpu.VMEM((2, HALF, N_PER), jnp.float32),
            pltpu.SemaphoreType.DMA((8,)),
            pltpu.SemaphoreType.DMA((8,)),
            pltpu.SemaphoreType.DMA((2,)),
            pltpu.SemaphoreType.DMA((2,)),
        ],
        compiler_params=pltpu.CompilerParams(
            collective_id=0,
            vmem_limit_bytes=60 * 1024 * 1024,
        ),
    )(xq, w_mat, s)


# device time: 111325 ns/iter; 1.0130x vs baseline; 1.0130x over previous
import jax
import jax.numpy as jnp
from jax import lax
from jax.experimental import pallas as pl
from jax.experimental.pallas import tpu as pltpu

N_DEV = 4
M_PER = 1024
K = 4096
N_PER = 2048
HALF = M_PER // 2
QUART = M_PER // 4
WCHUNK = 256

SLOT_L = 0
SLOT_R = 1
SLOT_O = 2


def _body(x_ref, w_hbm, s_ref, out_hbm,
          xq8, xg, wq, wstage, ostage,
          send_sems, recv_sems, wdma_sems, out_sems):
    my = lax.axis_index("i")
    left = (my + N_DEV - 1) % N_DEV
    right = (my + 1) % N_DEV
    opp = (my + 2) % N_DEV

    n_wchunks = N_PER // WCHUNK
    col0 = my * N_PER
    wdmas = []
    for c in range(2):
        d = pltpu.make_async_copy(
            w_hbm.at[:, pl.ds(col0 + c * WCHUNK, WCHUNK)],
            wstage.at[c % 2], wdma_sems.at[c % 2],
        )
        d.start()
        wdmas.append(d)

    xq8[...] = x_ref[...].astype(jnp.float8_e4m3fn)

    def rdma(src, dst, si, ri, dev):
        return pltpu.make_async_remote_copy(
            src_ref=src, dst_ref=dst,
            send_sem=send_sems.at[si], recv_sem=recv_sems.at[ri],
            device_id=(dev,), device_id_type=pl.DeviceIdType.MESH,
        )

    a = pl.ds(0, HALF)
    b = pl.ds(HALF, HALF)
    s_myA_r = rdma(xq8.at[a], xg.at[SLOT_L, a], 0, 0, right)
    s_myB_r = rdma(xq8.at[b], xg.at[SLOT_L, b], 1, 1, right)
    s_myB_l = rdma(xq8.at[b], xg.at[SLOT_R, b], 2, 2, left)
    s_myA_l = rdma(xq8.at[a], xg.at[SLOT_R, a], 3, 3, left)
    s_myA_r.start()
    s_myB_r.start()
    s_myB_l.start()
    s_myA_l.start()

    r_leftA = rdma(xq8.at[a], xg.at[SLOT_L, a], 0, 0, left)
    r_leftB = rdma(xq8.at[b], xg.at[SLOT_L, b], 1, 1, left)
    r_rightB = rdma(xq8.at[b], xg.at[SLOT_R, b], 2, 2, right)
    r_rightA = rdma(xq8.at[a], xg.at[SLOT_R, a], 3, 3, right)

    q = [pl.ds(i * QUART, QUART) for i in range(4)]
    s_fwd_r = [rdma(xg.at[SLOT_L, q[i]], xg.at[SLOT_O, q[i]], 4 + i, 4 + i, right)
               for i in range(2)]
    s_fwd_l = [rdma(xg.at[SLOT_R, q[i]], xg.at[SLOT_O, q[i]], 4 + i, 4 + i, left)
               for i in range(2, 4)]
    r_opp = [rdma(xg.at[SLOT_O, q[i]], xg.at[SLOT_O, q[i]], 4 + i, 4 + i,
                  left if i < 2 else right)
             for i in range(4)]

    for c in range(n_wchunks):
        wdmas[c].wait()
        if c + 2 < n_wchunks:
            d = pltpu.make_async_copy(
                w_hbm.at[:, pl.ds(col0 + (c + 2) * WCHUNK, WCHUNK)],
                wstage.at[(c + 2) % 2], wdma_sems.at[(c + 2) % 2],
            )
            d.start()
            wdmas.append(d)
        wq[:, pl.ds(c * WCHUNK, WCHUNK)] = wstage[c % 2].astype(jnp.float8_e5m2)

    s = s_ref[0]
    pending = [None, None]
    state = {"blk": 0}

    def block(x_block, out_row, rows):
        buf = state["blk"] % 2
        state["blk"] += 1
        if pending[buf] is not None:
            pending[buf].wait()
        acc = jnp.dot(x_block, wq[...], preferred_element_type=jnp.float32)
        y = acc * s
        z = y * (1.0 / (1.0 + jnp.exp(-jnp.clip(y, -60.0, 60.0))))
        ostage[buf, pl.ds(0, rows)] = z
        d = pltpu.make_async_copy(
            ostage.at[buf, pl.ds(0, rows)],
            out_hbm.at[pl.ds(out_row, rows)],
            out_sems.at[buf],
        )
        d.start()
        pending[buf] = d

    block(xq8[0:HALF, :], my * M_PER, HALF)
    block(xq8[HALF:M_PER, :], my * M_PER + HALF, HALF)

    r_leftA.wait_recv()
    for d in s_fwd_r:
        d.start()
    r_rightB.wait_recv()
    for d in s_fwd_l:
        d.start()
    block(xg[SLOT_L, 0:HALF, :], left * M_PER, HALF)
    block(xg[SLOT_R, HALF:M_PER, :], right * M_PER + HALF, HALF)

    r_leftB.wait_recv()
    block(xg[SLOT_L, HALF:M_PER, :], left * M_PER + HALF, HALF)
    r_rightA.wait_recv()
    block(xg[SLOT_R, 0:HALF, :], right * M_PER, HALF)

    for i in (0, 2, 1, 3):
        r_opp[i].wait_recv()
        block(xg[SLOT_O, i * QUART:(i + 1) * QUART, :],
              opp * M_PER + i * QUART, QUART)

    for d in pending:
        if d is not None:
            d.wait()
    for d in [s_myA_r, s_myB_r, s_myB_l, s_myA_l] + s_fwd_r + s_fwd_l:
        d.wait_send()


def kernel(x, w_mat, scale_x, scale_w):
    if x.dtype != jnp.float32:
        x = x.astype(jnp.float32)
    s = (scale_x * scale_w).astype(jnp.float32)

    return pl.pallas_call(
        _body,
        out_shape=jax.ShapeDtypeStruct((N_DEV * M_PER, N_PER), jnp.float32),
        in_specs=[
            pl.BlockSpec(memory_space=pltpu.MemorySpace.VMEM),
            pl.BlockSpec(memory_space=pltpu.MemorySpace.HBM),
            pl.BlockSpec(memory_space=pltpu.MemorySpace.SMEM),
        ],
        out_specs=pl.BlockSpec(memory_space=pltpu.MemorySpace.HBM),
        scratch_shapes=[
            pltpu.VMEM((M_PER, K), jnp.float8_e4m3fn),
            pltpu.VMEM((3, M_PER, K), jnp.float8_e4m3fn),
            pltpu.VMEM((K, N_PER), jnp.float8_e5m2),
            pltpu.VMEM((2, K, WCHUNK), jnp.float32),
            pltpu.VMEM((2, HALF, N_PER), jnp.float32),
            pltpu.SemaphoreType.DMA((8,)),
            pltpu.SemaphoreType.DMA((8,)),
            pltpu.SemaphoreType.DMA((2,)),
            pltpu.SemaphoreType.DMA((2,)),
        ],
        compiler_params=pltpu.CompilerParams(
            vmem_limit_bytes=60 * 1024 * 1024,
        ),
    )(x, w_mat, s)


# device time: 103285 ns/iter; 1.0918x vs baseline; 1.0778x over previous
import jax
import jax.numpy as jnp
from jax import lax
from jax.experimental import pallas as pl
from jax.experimental.pallas import tpu as pltpu

N_DEV = 4
M_PER = 1024
K = 4096
N_PER = 2048
HALF = M_PER // 2
QUART = M_PER // 4
WCHUNK = 256

SLOT_L = 0
SLOT_R = 1
SLOT_O = 2


def _body(x_ref, w_hbm, s_ref, out_hbm,
          xq8, xg, wq, wstage, ostage,
          send_sems, recv_sems, wdma_sems, out_sems):
    my = lax.axis_index("i")
    left = (my + N_DEV - 1) % N_DEV
    right = (my + 1) % N_DEV
    opp = (my + 2) % N_DEV

    barrier = pltpu.get_barrier_semaphore()
    for nbr in (left, right):
        pl.semaphore_signal(
            barrier, inc=1,
            device_id=(nbr,), device_id_type=pl.DeviceIdType.MESH,
        )
    pl.semaphore_wait(barrier, 2)

    n_wchunks = N_PER // WCHUNK
    col0 = my * N_PER
    wdmas = []
    for c in range(2):
        d = pltpu.make_async_copy(
            w_hbm.at[:, pl.ds(col0 + c * WCHUNK, WCHUNK)],
            wstage.at[c % 2], wdma_sems.at[c % 2],
        )
        d.start()
        wdmas.append(d)

    def rdma(src, dst, si, ri, dev):
        return pltpu.make_async_remote_copy(
            src_ref=src, dst_ref=dst,
            send_sem=send_sems.at[si], recv_sem=recv_sems.at[ri],
            device_id=(dev,), device_id_type=pl.DeviceIdType.MESH,
        )

    a = pl.ds(0, HALF)
    b = pl.ds(HALF, HALF)
    s_myA_r = rdma(xq8.at[a], xg.at[SLOT_L, a], 0, 0, right)
    s_myB_r = rdma(xq8.at[b], xg.at[SLOT_L, b], 1, 1, right)
    s_myB_l = rdma(xq8.at[b], xg.at[SLOT_R, b], 2, 2, left)
    s_myA_l = rdma(xq8.at[a], xg.at[SLOT_R, a], 3, 3, left)
    xq8[a, :] = x_ref[0:HALF, :].astype(jnp.float8_e4m3fn)
    s_myA_r.start()
    xq8[b, :] = x_ref[HALF:M_PER, :].astype(jnp.float8_e4m3fn)
    s_myB_l.start()
    s_myB_r.start()
    s_myA_l.start()

    r_leftA = rdma(xq8.at[a], xg.at[SLOT_L, a], 0, 0, left)
    r_leftB = rdma(xq8.at[b], xg.at[SLOT_L, b], 1, 1, left)
    r_rightB = rdma(xq8.at[b], xg.at[SLOT_R, b], 2, 2, right)
    r_rightA = rdma(xq8.at[a], xg.at[SLOT_R, a], 3, 3, right)

    q = [pl.ds(i * QUART, QUART) for i in range(4)]
    s_fwd_r = [rdma(xg.at[SLOT_L, q[i]], xg.at[SLOT_O, q[i]], 4 + i, 4 + i, right)
               for i in range(2)]
    s_fwd_l = [rdma(xg.at[SLOT_R, q[i]], xg.at[SLOT_O, q[i]], 4 + i, 4 + i, left)
               for i in range(2, 4)]
    r_opp = [rdma(xg.at[SLOT_O, q[i]], xg.at[SLOT_O, q[i]], 4 + i, 4 + i,
                  left if i < 2 else right)
             for i in range(4)]

    for c in range(n_wchunks):
        wdmas[c].wait()
        if c + 2 < n_wchunks:
            d = pltpu.make_async_copy(
                w_hbm.at[:, pl.ds(col0 + (c + 2) * WCHUNK, WCHUNK)],
                wstage.at[(c + 2) % 2], wdma_sems.at[(c + 2) % 2],
            )
            d.start()
            wdmas.append(d)
        wq[:, pl.ds(c * WCHUNK, WCHUNK)] = wstage[c % 2].astype(jnp.float8_e5m2)

    s = s_ref[0]
    pending = [None, None]
    state = {"blk": 0}

    def block(x_block, out_row, rows):
        buf = state["blk"] % 2
        state["blk"] += 1
        if pending[buf] is not None:
            pending[buf].wait()
        acc = jnp.dot(x_block, wq[...], preferred_element_type=jnp.float32)
        y = acc * s
        z = y * (1.0 / (1.0 + jnp.exp(-jnp.clip(y, -60.0, 60.0))))
        ostage[buf, pl.ds(0, rows)] = z.astype(jnp.bfloat16)
        d = pltpu.make_async_copy(
            ostage.at[buf, pl.ds(0, rows)],
            out_hbm.at[pl.ds(out_row, rows)],
            out_sems.at[buf],
        )
        d.start()
        pending[buf] = d

    block(xq8[0:HALF, :], my * M_PER, HALF)
    block(xq8[HALF:M_PER, :], my * M_PER + HALF, HALF)

    r_leftA.wait_recv()
    for d in s_fwd_r:
        d.start()
    r_rightB.wait_recv()
    for d in s_fwd_l:
        d.start()
    block(xg[SLOT_L, 0:HALF, :], left * M_PER, HALF)
    block(xg[SLOT_R, HALF:M_PER, :], right * M_PER + HALF, HALF)

    r_leftB.wait_recv()
    block(xg[SLOT_L, HALF:M_PER, :], left * M_PER + HALF, HALF)
    r_rightA.wait_recv()
    block(xg[SLOT_R, 0:HALF, :], right * M_PER, HALF)

    for i in (0, 2, 1, 3):
        r_opp[i].wait_recv()
        block(xg[SLOT_O, i * QUART:(i + 1) * QUART, :],
              opp * M_PER + i * QUART, QUART)

    for d in pending:
        if d is not None:
            d.wait()
    for d in [s_myA_r, s_myB_r, s_myB_l, s_myA_l] + s_fwd_r + s_fwd_l:
        d.wait_send()


def kernel(x, w_mat, scale_x, scale_w):
    if x.dtype != jnp.float32:
        x = x.astype(jnp.float32)
    s = (scale_x * scale_w).astype(jnp.float32)

    out16 = pl.pallas_call(
        _body,
        out_shape=jax.ShapeDtypeStruct((N_DEV * M_PER, N_PER), jnp.bfloat16),
        in_specs=[
            pl.BlockSpec(memory_space=pltpu.MemorySpace.VMEM),
            pl.BlockSpec(memory_space=pltpu.MemorySpace.HBM),
            pl.BlockSpec(memory_space=pltpu.MemorySpace.SMEM),
        ],
        out_specs=pl.BlockSpec(memory_space=pltpu.MemorySpace.HBM),
        scratch_shapes=[
            pltpu.VMEM((M_PER, K), jnp.float8_e4m3fn),
            pltpu.VMEM((3, M_PER, K), jnp.float8_e4m3fn),
            pltpu.VMEM((K, N_PER), jnp.float8_e5m2),
            pltpu.VMEM((2, K, WCHUNK), jnp.float32),
            pltpu.VMEM((2, HALF, N_PER), jnp.bfloat16),
            pltpu.SemaphoreType.DMA((8,)),
            pltpu.SemaphoreType.DMA((8,)),
            pltpu.SemaphoreType.DMA((2,)),
            pltpu.SemaphoreType.DMA((2,)),
        ],
        compiler_params=pltpu.CompilerParams(
            collective_id=0,
            vmem_limit_bytes=60 * 1024 * 1024,
        ),
    )(x, w_mat, s)
    return out16.astype(jnp.float32)


# device time: 97773 ns/iter; 1.1534x vs baseline; 1.0564x over previous
import jax
import jax.numpy as jnp
from jax import lax
from jax.experimental import pallas as pl
from jax.experimental.pallas import tpu as pltpu

N_DEV = 4
M_PER = 1024
K = 4096
N_PER = 2048
HALF = M_PER // 2
QUART = M_PER // 4
WCHUNK = 256

SLOT_L = 0
SLOT_R = 1
SLOT_O = 2


def _body(x_ref, w_hbm, s_ref, out_hbm,
          xq8, xg, wq, wstage, ostage,
          send_sems, recv_sems, wdma_sems, out_sems):
    my = lax.axis_index("i")
    left = (my + N_DEV - 1) % N_DEV
    right = (my + 1) % N_DEV
    opp = (my + 2) % N_DEV

    barrier = pltpu.get_barrier_semaphore()
    for nbr in (left, right):
        pl.semaphore_signal(
            barrier, inc=1,
            device_id=(nbr,), device_id_type=pl.DeviceIdType.MESH,
        )
    pl.semaphore_wait(barrier, 2)

    n_wchunks = N_PER // WCHUNK
    col0 = my * N_PER
    wdmas = []
    for c in range(2):
        d = pltpu.make_async_copy(
            w_hbm.at[:, pl.ds(col0 + c * WCHUNK, WCHUNK)],
            wstage.at[c % 2], wdma_sems.at[c % 2],
        )
        d.start()
        wdmas.append(d)

    def rdma(src, dst, si, ri, dev):
        return pltpu.make_async_remote_copy(
            src_ref=src, dst_ref=dst,
            send_sem=send_sems.at[si], recv_sem=recv_sems.at[ri],
            device_id=(dev,), device_id_type=pl.DeviceIdType.MESH,
        )

    a = pl.ds(0, HALF)
    b = pl.ds(HALF, HALF)
    s_myA_r = rdma(xq8.at[a], xg.at[SLOT_L, a], 0, 0, right)
    s_myB_r = rdma(xq8.at[b], xg.at[SLOT_L, b], 1, 1, right)
    s_myB_l = rdma(xq8.at[b], xg.at[SLOT_R, b], 2, 2, left)
    s_myA_l = rdma(xq8.at[a], xg.at[SLOT_R, a], 3, 3, left)
    xq8[a, :] = x_ref[0:HALF, :].astype(jnp.float8_e4m3fn)
    s_myA_r.start()
    xq8[b, :] = x_ref[HALF:M_PER, :].astype(jnp.float8_e4m3fn)
    s_myB_l.start()
    s_myB_r.start()
    s_myA_l.start()

    r_leftA = rdma(xq8.at[a], xg.at[SLOT_L, a], 0, 0, left)
    r_leftB = rdma(xq8.at[b], xg.at[SLOT_L, b], 1, 1, left)
    r_rightB = rdma(xq8.at[b], xg.at[SLOT_R, b], 2, 2, right)
    r_rightA = rdma(xq8.at[a], xg.at[SLOT_R, a], 3, 3, right)

    q = [pl.ds(i * QUART, QUART) for i in range(4)]
    s_fwd_r = [rdma(xg.at[SLOT_L, q[i]], xg.at[SLOT_O, q[i]], 4 + i, 4 + i, right)
               for i in range(2)]
    s_fwd_l = [rdma(xg.at[SLOT_R, q[i]], xg.at[SLOT_O, q[i]], 4 + i, 4 + i, left)
               for i in range(2, 4)]
    r_opp = [rdma(xg.at[SLOT_O, q[i]], xg.at[SLOT_O, q[i]], 4 + i, 4 + i,
                  left if i < 2 else right)
             for i in range(4)]

    for c in range(n_wchunks):
        wdmas[c].wait()
        if c + 2 < n_wchunks:
            d = pltpu.make_async_copy(
                w_hbm.at[:, pl.ds(col0 + (c + 2) * WCHUNK, WCHUNK)],
                wstage.at[(c + 2) % 2], wdma_sems.at[(c + 2) % 2],
            )
            d.start()
            wdmas.append(d)
        wq[:, pl.ds(c * WCHUNK, WCHUNK)] = wstage[c % 2].astype(jnp.float8_e5m2)

    s = s_ref[0]
    pending = [None, None]
    state = {"blk": 0}

    def block(x_block, out_row, rows):
        buf = state["blk"] % 2
        state["blk"] += 1
        if pending[buf] is not None:
            pending[buf].wait()
        acc = jnp.dot(x_block, wq[...], preferred_element_type=jnp.float32)
        y = acc * s
        z = y * (1.0 / (1.0 + jnp.exp(-y)))
        ostage[buf, pl.ds(0, rows)] = z.astype(jnp.bfloat16)
        d = pltpu.make_async_copy(
            ostage.at[buf, pl.ds(0, rows)],
            out_hbm.at[pl.ds(out_row, rows)],
            out_sems.at[buf],
        )
        d.start()
        pending[buf] = d

    block(xq8[0:HALF, :], my * M_PER, HALF)
    block(xq8[HALF:M_PER, :], my * M_PER + HALF, HALF)

    r_leftA.wait_recv()
    for d in s_fwd_r:
        d.start()
    r_rightB.wait_recv()
    for d in s_fwd_l:
        d.start()
    block(xg[SLOT_L, 0:HALF, :], left * M_PER, HALF)
    block(xg[SLOT_R, HALF:M_PER, :], right * M_PER + HALF, HALF)

    r_leftB.wait_recv()
    block(xg[SLOT_L, HALF:M_PER, :], left * M_PER + HALF, HALF)
    r_rightA.wait_recv()
    block(xg[SLOT_R, 0:HALF, :], right * M_PER, HALF)

    for i in (0, 2, 1, 3):
        r_opp[i].wait_recv()
        block(xg[SLOT_O, i * QUART:(i + 1) * QUART, :],
              opp * M_PER + i * QUART, QUART)

    for d in pending:
        if d is not None:
            d.wait()
    for d in [s_myA_r, s_myB_r, s_myB_l, s_myA_l] + s_fwd_r + s_fwd_l:
        d.wait_send()


def kernel(x, w_mat, scale_x, scale_w):
    if x.dtype != jnp.float32:
        x = x.astype(jnp.float32)
    s = (scale_x * scale_w).astype(jnp.float32)

    return pl.pallas_call(
        _body,
        out_shape=jax.ShapeDtypeStruct((N_DEV * M_PER, N_PER), jnp.bfloat16),
        in_specs=[
            pl.BlockSpec(memory_space=pltpu.MemorySpace.VMEM),
            pl.BlockSpec(memory_space=pltpu.MemorySpace.HBM),
            pl.BlockSpec(memory_space=pltpu.MemorySpace.SMEM),
        ],
        out_specs=pl.BlockSpec(memory_space=pltpu.MemorySpace.HBM),
        scratch_shapes=[
            pltpu.VMEM((M_PER, K), jnp.float8_e4m3fn),
            pltpu.VMEM((3, M_PER, K), jnp.float8_e4m3fn),
            pltpu.VMEM((K, N_PER), jnp.float8_e5m2),
            pltpu.VMEM((2, K, WCHUNK), jnp.float32),
            pltpu.VMEM((2, HALF, N_PER), jnp.bfloat16),
            pltpu.SemaphoreType.DMA((8,)),
            pltpu.SemaphoreType.DMA((8,)),
            pltpu.SemaphoreType.DMA((2,)),
            pltpu.SemaphoreType.DMA((2,)),
        ],
        compiler_params=pltpu.CompilerParams(
            collective_id=0,
            vmem_limit_bytes=60 * 1024 * 1024,
        ),
    )(x, w_mat, s)


# device time: 95613 ns/iter; 1.1794x vs baseline; 1.0226x over previous
import jax
import jax.numpy as jnp
from jax import lax
from jax.experimental import pallas as pl
from jax.experimental.pallas import tpu as pltpu

N_DEV = 4
M_PER = 1024
K = 4096
N_PER = 2048
HALF = M_PER // 2
QUART = M_PER // 4
WCHUNK = 256

SLOT_L = 0
SLOT_R = 1
SLOT_O = 2


def _body(x_hbm, w_hbm, s_ref, out_hbm,
          xstage, xq8, xg, wq, wstage, ostage,
          send_sems, recv_sems, xdma_sems, wdma_sems, out_sems):
    my = lax.axis_index("i")
    left = (my + N_DEV - 1) % N_DEV
    right = (my + 1) % N_DEV
    opp = (my + 2) % N_DEV

    xdmas = []
    for h in range(2):
        d = pltpu.make_async_copy(
            x_hbm.at[pl.ds(h * HALF, HALF)], xstage.at[h], xdma_sems.at[h],
        )
        d.start()
        xdmas.append(d)
    n_wchunks = N_PER // WCHUNK
    col0 = my * N_PER
    wdmas = []
    for c in range(2):
        d = pltpu.make_async_copy(
            w_hbm.at[:, pl.ds(col0 + c * WCHUNK, WCHUNK)],
            wstage.at[c % 2], wdma_sems.at[c % 2],
        )
        d.start()
        wdmas.append(d)

    barrier = pltpu.get_barrier_semaphore()
    for nbr in (left, right):
        pl.semaphore_signal(
            barrier, inc=1,
            device_id=(nbr,), device_id_type=pl.DeviceIdType.MESH,
        )
    pl.semaphore_wait(barrier, 2)

    def rdma(src, dst, si, ri, dev):
        return pltpu.make_async_remote_copy(
            src_ref=src, dst_ref=dst,
            send_sem=send_sems.at[si], recv_sem=recv_sems.at[ri],
            device_id=(dev,), device_id_type=pl.DeviceIdType.MESH,
        )

    a = pl.ds(0, HALF)
    b = pl.ds(HALF, HALF)
    s_myA_r = rdma(xq8.at[a], xg.at[SLOT_L, a], 0, 0, right)
    s_myB_r = rdma(xq8.at[b], xg.at[SLOT_L, b], 1, 1, right)
    s_myB_l = rdma(xq8.at[b], xg.at[SLOT_R, b], 2, 2, left)
    s_myA_l = rdma(xq8.at[a], xg.at[SLOT_R, a], 3, 3, left)
    xdmas[0].wait()
    xq8[a, :] = xstage[0].astype(jnp.float8_e4m3fn)
    s_myA_r.start()
    xdmas[1].wait()
    xq8[b, :] = xstage[1].astype(jnp.float8_e4m3fn)
    s_myB_l.start()
    s_myB_r.start()
    s_myA_l.start()

    r_leftA = rdma(xq8.at[a], xg.at[SLOT_L, a], 0, 0, left)
    r_leftB = rdma(xq8.at[b], xg.at[SLOT_L, b], 1, 1, left)
    r_rightB = rdma(xq8.at[b], xg.at[SLOT_R, b], 2, 2, right)
    r_rightA = rdma(xq8.at[a], xg.at[SLOT_R, a], 3, 3, right)

    q = [pl.ds(i * QUART, QUART) for i in range(4)]
    s_fwd_r = [rdma(xg.at[SLOT_L, q[i]], xg.at[SLOT_O, q[i]], 4 + i, 4 + i, right)
               for i in range(2)]
    s_fwd_l = [rdma(xg.at[SLOT_R, q[i]], xg.at[SLOT_O, q[i]], 4 + i, 4 + i, left)
               for i in range(2, 4)]
    r_opp = [rdma(xg.at[SLOT_O, q[i]], xg.at[SLOT_O, q[i]], 4 + i, 4 + i,
                  left if i < 2 else right)
             for i in range(4)]

    for c in range(n_wchunks):
        wdmas[c].wait()
        if c + 2 < n_wchunks:
            d = pltpu.make_async_copy(
                w_hbm.at[:, pl.ds(col0 + (c + 2) * WCHUNK, WCHUNK)],
                wstage.at[(c + 2) % 2], wdma_sems.at[(c + 2) % 2],
            )
            d.start()
            wdmas.append(d)
        wq[:, pl.ds(c * WCHUNK, WCHUNK)] = wstage[c % 2].astype(jnp.float8_e5m2)

    s = s_ref[0]
    pending = [None, None]
    state = {"blk": 0}

    def block(x_block, out_row, rows):
        buf = state["blk"] % 2
        state["blk"] += 1
        if pending[buf] is not None:
            pending[buf].wait()
        acc = jnp.dot(x_block, wq[...], preferred_element_type=jnp.float32)
        y = acc * s
        z = y * (1.0 / (1.0 + jnp.exp(-y)))
        ostage[buf, pl.ds(0, rows)] = z.astype(jnp.bfloat16)
        d = pltpu.make_async_copy(
            ostage.at[buf, pl.ds(0, rows)],
            out_hbm.at[pl.ds(out_row, rows)],
            out_sems.at[buf],
        )
        d.start()
        pending[buf] = d

    block(xq8[0:HALF, :], my * M_PER, HALF)
    block(xq8[HALF:M_PER, :], my * M_PER + HALF, HALF)

    r_leftA.wait_recv()
    for d in s_fwd_r:
        d.start()
    r_rightB.wait_recv()
    for d in s_fwd_l:
        d.start()
    block(xg[SLOT_L, 0:HALF, :], left * M_PER, HALF)
    block(xg[SLOT_R, HALF:M_PER, :], right * M_PER + HALF, HALF)

    r_leftB.wait_recv()
    block(xg[SLOT_L, HALF:M_PER, :], left * M_PER + HALF, HALF)
    r_rightA.wait_recv()
    block(xg[SLOT_R, 0:HALF, :], right * M_PER, HALF)

    for i in (0, 2, 1, 3):
        r_opp[i].wait_recv()
        block(xg[SLOT_O, i * QUART:(i + 1) * QUART, :],
              opp * M_PER + i * QUART, QUART)

    for d in pending:
        if d is not None:
            d.wait()
    for d in [s_myA_r, s_myB_r, s_myB_l, s_myA_l] + s_fwd_r + s_fwd_l:
        d.wait_send()


def kernel(x, w_mat, scale_x, scale_w):
    if x.dtype != jnp.float32:
        x = x.astype(jnp.float32)
    s = (scale_x * scale_w).astype(jnp.float32)

    return pl.pallas_call(
        _body,
        out_shape=jax.ShapeDtypeStruct((N_DEV * M_PER, N_PER), jnp.bfloat16),
        in_specs=[
            pl.BlockSpec(memory_space=pltpu.MemorySpace.HBM),
            pl.BlockSpec(memory_space=pltpu.MemorySpace.HBM),
            pl.BlockSpec(memory_space=pltpu.MemorySpace.SMEM),
        ],
        out_specs=pl.BlockSpec(memory_space=pltpu.MemorySpace.HBM),
        scratch_shapes=[
            pltpu.VMEM((2, HALF, K), jnp.float32),
            pltpu.VMEM((M_PER, K), jnp.float8_e4m3fn),
            pltpu.VMEM((3, M_PER, K), jnp.float8_e4m3fn),
            pltpu.VMEM((K, N_PER), jnp.float8_e5m2),
            pltpu.VMEM((2, K, WCHUNK), jnp.float32),
            pltpu.VMEM((2, HALF, N_PER), jnp.bfloat16),
            pltpu.SemaphoreType.DMA((8,)),
            pltpu.SemaphoreType.DMA((8,)),
            pltpu.SemaphoreType.DMA((2,)),
            pltpu.SemaphoreType.DMA((2,)),
            pltpu.SemaphoreType.DMA((2,)),
        ],
        compiler_params=pltpu.CompilerParams(
            collective_id=0,
            vmem_limit_bytes=60 * 1024 * 1024,
        ),
    )(x, w_mat, s)


# device time: 94345 ns/iter; 1.1953x vs baseline; 1.0134x over previous
import jax
import jax.numpy as jnp
from jax import lax
from jax.experimental import pallas as pl
from jax.experimental.pallas import tpu as pltpu

N_DEV = 4
M_PER = 1024
K = 4096
N_PER = 2048
HALF = M_PER // 2
QUART = M_PER // 4
WCHUNK = 256

SLOT_L = 0
SLOT_R = 1
SLOT_O = 2


def _body(x_hbm, w_hbm, s_ref, out_hbm,
          xstage, xq8, xg, wq, wstage, ostage,
          send_sems, recv_sems, xdma_sems, wdma_sems, out_sems):
    my = lax.axis_index("i")
    left = (my + N_DEV - 1) % N_DEV
    right = (my + 1) % N_DEV
    opp = (my + 2) % N_DEV

    xdmas = []
    for h, qi in ((0, 0), (1, 0), (0, 1), (1, 1)):
        d = pltpu.make_async_copy(
            x_hbm.at[pl.ds(h * HALF + qi * QUART, QUART)],
            xstage.at[h, pl.ds(qi * QUART, QUART)],
            xdma_sems.at[2 * h + qi],
        )
        d.start()
        xdmas.append(d)
    n_wchunks = N_PER // WCHUNK
    col0 = my * N_PER
    wdmas = []
    for c in range(2):
        d = pltpu.make_async_copy(
            w_hbm.at[:, pl.ds(col0 + c * WCHUNK, WCHUNK)],
            wstage.at[c % 2], wdma_sems.at[c % 2],
        )
        d.start()
        wdmas.append(d)

    barrier = pltpu.get_barrier_semaphore()
    for nbr in (left, right):
        pl.semaphore_signal(
            barrier, inc=1,
            device_id=(nbr,), device_id_type=pl.DeviceIdType.MESH,
        )
    pl.semaphore_wait(barrier, 2)

    def rdma(src, dst, si, ri, dev):
        return pltpu.make_async_remote_copy(
            src_ref=src, dst_ref=dst,
            send_sem=send_sems.at[si], recv_sem=recv_sems.at[ri],
            device_id=(dev,), device_id_type=pl.DeviceIdType.MESH,
        )

    a = pl.ds(0, HALF)
    b = pl.ds(HALF, HALF)
    aq = [pl.ds(i * QUART, QUART) for i in range(2)]
    bq = [pl.ds(HALF + i * QUART, QUART) for i in range(2)]
    s_myAq_r = [rdma(xq8.at[aq[i]], xg.at[SLOT_L, aq[i]], i, i, right)
                for i in range(2)]
    s_myB_r = rdma(xq8.at[b], xg.at[SLOT_L, b], 2, 2, right)
    s_myBq_l = [rdma(xq8.at[bq[i]], xg.at[SLOT_R, bq[i]], 3 + i, 3 + i, left)
                for i in range(2)]
    s_myA_l = rdma(xq8.at[a], xg.at[SLOT_R, a], 5, 5, left)
    xdmas[0].wait()
    xq8[aq[0], :] = xstage[0, 0:QUART].astype(jnp.float8_e4m3fn)
    s_myAq_r[0].start()
    xdmas[1].wait()
    xq8[bq[0], :] = xstage[1, 0:QUART].astype(jnp.float8_e4m3fn)
    s_myBq_l[0].start()
    xdmas[2].wait()
    xq8[aq[1], :] = xstage[0, QUART:HALF].astype(jnp.float8_e4m3fn)
    s_myAq_r[1].start()
    xdmas[3].wait()
    xq8[bq[1], :] = xstage[1, QUART:HALF].astype(jnp.float8_e4m3fn)
    s_myBq_l[1].start()
    s_myB_r.start()
    s_myA_l.start()

    r_leftAq = [rdma(xq8.at[aq[i]], xg.at[SLOT_L, aq[i]], i, i, left)
                for i in range(2)]
    r_leftB = rdma(xq8.at[b], xg.at[SLOT_L, b], 2, 2, left)
    r_rightBq = [rdma(xq8.at[bq[i]], xg.at[SLOT_R, bq[i]], 3 + i, 3 + i, right)
                 for i in range(2)]
    r_rightA = rdma(xq8.at[a], xg.at[SLOT_R, a], 5, 5, right)

    q = [pl.ds(i * QUART, QUART) for i in range(4)]
    s_fwd_r = [rdma(xg.at[SLOT_L, q[i]], xg.at[SLOT_O, q[i]], 6 + i, 6 + i, right)
               for i in range(2)]
    s_fwd_l = [rdma(xg.at[SLOT_R, q[i]], xg.at[SLOT_O, q[i]], 6 + i, 6 + i, left)
               for i in range(2, 4)]
    r_opp = [rdma(xg.at[SLOT_O, q[i]], xg.at[SLOT_O, q[i]], 6 + i, 6 + i,
                  left if i < 2 else right)
             for i in range(4)]

    for c in range(n_wchunks):
        wdmas[c].wait()
        if c + 2 < n_wchunks:
            d = pltpu.make_async_copy(
                w_hbm.at[:, pl.ds(col0 + (c + 2) * WCHUNK, WCHUNK)],
                wstage.at[(c + 2) % 2], wdma_sems.at[(c + 2) % 2],
            )
            d.start()
            wdmas.append(d)
        wq[:, pl.ds(c * WCHUNK, WCHUNK)] = wstage[c % 2].astype(jnp.float8_e5m2)

    s = s_ref[0]
    pending = [None, None]
    state = {"blk": 0}

    def block(x_block, out_row, rows):
        buf = state["blk"] % 2
        state["blk"] += 1
        if pending[buf] is not None:
            pending[buf].wait()
        acc = jnp.dot(x_block, wq[...], preferred_element_type=jnp.float32)
        y = acc * s
        z = y * (1.0 / (1.0 + jnp.exp(-y)))
        ostage[buf, pl.ds(0, rows)] = z.astype(jnp.bfloat16)
        d = pltpu.make_async_copy(
            ostage.at[buf, pl.ds(0, rows)],
            out_hbm.at[pl.ds(out_row, rows)],
            out_sems.at[buf],
        )
        d.start()
        pending[buf] = d

    block(xq8[0:HALF, :], my * M_PER, HALF)
    block(xq8[HALF:M_PER, :], my * M_PER + HALF, HALF)

    for d in r_leftAq:
        d.wait_recv()
    for d in s_fwd_r:
        d.start()
    for d in r_rightBq:
        d.wait_recv()
    for d in s_fwd_l:
        d.start()
    block(xg[SLOT_L, 0:HALF, :], left * M_PER, HALF)
    block(xg[SLOT_R, HALF:M_PER, :], right * M_PER + HALF, HALF)

    r_leftB.wait_recv()
    block(xg[SLOT_L, HALF:M_PER, :], left * M_PER + HALF, HALF)
    r_rightA.wait_recv()
    block(xg[SLOT_R, 0:HALF, :], right * M_PER, HALF)

    for i in (0, 2, 1, 3):
        r_opp[i].wait_recv()
        block(xg[SLOT_O, i * QUART:(i + 1) * QUART, :],
              opp * M_PER + i * QUART, QUART)

    for d in pending:
        if d is not None:
            d.wait()
    for d in s_myAq_r + [s_myB_r] + s_myBq_l + [s_myA_l] + s_fwd_r + s_fwd_l:
        d.wait_send()


def kernel(x, w_mat, scale_x, scale_w):
    if x.dtype != jnp.float32:
        x = x.astype(jnp.float32)
    s = (scale_x * scale_w).astype(jnp.float32)

    return pl.pallas_call(
        _body,
        out_shape=jax.ShapeDtypeStruct((N_DEV * M_PER, N_PER), jnp.bfloat16),
        in_specs=[
            pl.BlockSpec(memory_space=pltpu.MemorySpace.HBM),
            pl.BlockSpec(memory_space=pltpu.MemorySpace.HBM),
            pl.BlockSpec(memory_space=pltpu.MemorySpace.SMEM),
        ],
        out_specs=pl.BlockSpec(memory_space=pltpu.MemorySpace.HBM),
        scratch_shapes=[
            pltpu.VMEM((2, HALF, K), jnp.float32),
            pltpu.VMEM((M_PER, K), jnp.float8_e4m3fn),
            pltpu.VMEM((3, M_PER, K), jnp.float8_e4m3fn),
            pltpu.VMEM((K, N_PER), jnp.float8_e5m2),
            pltpu.VMEM((2, K, WCHUNK), jnp.float32),
            pltpu.VMEM((2, HALF, N_PER), jnp.bfloat16),
            pltpu.SemaphoreType.DMA((10,)),
            pltpu.SemaphoreType.DMA((10,)),
            pltpu.SemaphoreType.DMA((4,)),
            pltpu.SemaphoreType.DMA((2,)),
            pltpu.SemaphoreType.DMA((2,)),
        ],
        compiler_params=pltpu.CompilerParams(
            collective_id=0,
            vmem_limit_bytes=60 * 1024 * 1024,
        ),
    )(x, w_mat, s)
